# baseline (speedup 1.0000x reference)
import jax
import jax.numpy as jnp
from jax.experimental import pallas as pl
from jax.experimental.pallas import tpu as pltpu

_TILE = 12544


def _proj_kernel(x_ref, rm_ref, o_ref):
    o_ref[0] = jax.lax.dot_general(
        x_ref[0].astype(jnp.bfloat16),
        rm_ref[0].astype(jnp.bfloat16),
        dimension_numbers=(((1,), (0,)), ((), ())),
        preferred_element_type=jnp.float32,
    )


def kernel(inputs, random_matrices):
    n, c, h, w = inputs.shape
    hw = h * w
    m = random_matrices.shape[2]
    xt = inputs.reshape(n, c, hw).transpose(0, 2, 1)

    tile = _TILE if hw % _TILE == 0 else hw
    grid = (n, hw // tile)

    return pl.pallas_call(
        _proj_kernel,
        grid=grid,
        in_specs=[
            pl.BlockSpec((1, tile, c), lambda b, t: (b, t, 0)),
            pl.BlockSpec((1, c, m), lambda b, t: (b, 0, 0)),
        ],
        out_specs=pl.BlockSpec((1, tile, m), lambda b, t: (b, t, 0)),
        out_shape=jax.ShapeDtypeStruct((n, hw, m), jnp.float32),
        compiler_params=pltpu.CompilerParams(
            dimension_semantics=("parallel", "parallel"),
        ),
    )(xt, random_matrices)


# speedup vs baseline: 1.0062x; 1.0062x over previous
import jax
import jax.numpy as jnp
from jax.experimental import pallas as pl
from jax.experimental.pallas import tpu as pltpu

_TILE = 6272


def _proj_kernel(x_ref, rm_ref, o_ref):
    o_ref[0] = jax.lax.dot_general(
        x_ref[0].astype(jnp.bfloat16),
        rm_ref[0].astype(jnp.bfloat16),
        dimension_numbers=(((1,), (0,)), ((), ())),
        preferred_element_type=jnp.float32,
    )


def kernel(inputs, random_matrices):
    n, c, h, w = inputs.shape
    hw = h * w
    m = random_matrices.shape[2]
    xt = inputs.reshape(n, c, hw).transpose(0, 2, 1)

    tile = _TILE if hw % _TILE == 0 else hw
    grid = (n, hw // tile)

    return pl.pallas_call(
        _proj_kernel,
        grid=grid,
        in_specs=[
            pl.BlockSpec((1, tile, c), lambda b, t: (b, t, 0)),
            pl.BlockSpec((1, c, m), lambda b, t: (b, 0, 0)),
        ],
        out_specs=pl.BlockSpec((1, tile, m), lambda b, t: (b, t, 0)),
        out_shape=jax.ShapeDtypeStruct((n, hw, m), jnp.float32),
        compiler_params=pltpu.CompilerParams(
            dimension_semantics=("parallel", "parallel"),
        ),
    )(xt, random_matrices)


# speedup vs baseline: 1.0082x; 1.0020x over previous
import jax
import jax.numpy as jnp
from jax.experimental import pallas as pl
from jax.experimental.pallas import tpu as pltpu

_TILE = 7168


def _proj_kernel(x_ref, rm_ref, o_ref):
    o_ref[0] = jax.lax.dot_general(
        x_ref[0],
        rm_ref[0],
        dimension_numbers=(((1,), (0,)), ((), ())),
        precision=jax.lax.Precision.DEFAULT,
        preferred_element_type=jnp.float32,
    )


def kernel(inputs, random_matrices):
    n, c, h, w = inputs.shape
    hw = h * w
    m = random_matrices.shape[2]
    xt = inputs.reshape(n, c, hw).transpose(0, 2, 1)

    tile = _TILE if hw % _TILE == 0 else hw
    grid = (n, hw // tile)

    return pl.pallas_call(
        _proj_kernel,
        grid=grid,
        in_specs=[
            pl.BlockSpec((1, tile, c), lambda b, t: (b, t, 0)),
            pl.BlockSpec((1, c, m), lambda b, t: (b, 0, 0)),
        ],
        out_specs=pl.BlockSpec((1, tile, m), lambda b, t: (b, t, 0)),
        out_shape=jax.ShapeDtypeStruct((n, hw, m), jnp.float32),
        compiler_params=pltpu.CompilerParams(
            dimension_semantics=("parallel", "parallel"),
        ),
    )(xt, random_matrices)


# speedup vs baseline: 1.0085x; 1.0003x over previous
import jax
import jax.numpy as jnp
from jax.experimental import pallas as pl
from jax.experimental.pallas import tpu as pltpu

_TILE = 7168


def _proj_kernel(xa_ref, xb_ref, rm_ref, o_ref):
    half = xa_ref.shape[1]
    rm = rm_ref[0]
    for i, x_ref in enumerate((xa_ref, xb_ref)):
        o_ref[0, pl.ds(i * half, half), :] = jax.lax.dot_general(
            x_ref[0],
            rm,
            dimension_numbers=(((1,), (0,)), ((), ())),
            precision=jax.lax.Precision.DEFAULT,
            preferred_element_type=jnp.float32,
        )


def kernel(inputs, random_matrices):
    n, c, h, w = inputs.shape
    hw = h * w
    m = random_matrices.shape[2]
    xt = inputs.reshape(n, c, hw).transpose(0, 2, 1)

    tile = _TILE if hw % _TILE == 0 else hw
    grid = (n, hw // tile)

    return pl.pallas_call(
        _proj_kernel,
        grid=grid,
        in_specs=[
            pl.BlockSpec((1, tile // 2, c), lambda b, t: (b, 2 * t, 0)),
            pl.BlockSpec((1, tile // 2, c), lambda b, t: (b, 2 * t + 1, 0)),
            pl.BlockSpec((1, c, m), lambda b, t: (b, 0, 0)),
        ],
        out_specs=pl.BlockSpec((1, tile, m), lambda b, t: (b, t, 0)),
        out_shape=jax.ShapeDtypeStruct((n, hw, m), jnp.float32),
        compiler_params=pltpu.CompilerParams(
            dimension_semantics=("parallel", "parallel"),
        ),
    )(xt, xt, random_matrices)


# speedup vs baseline: 1.0115x; 1.0030x over previous
import jax
import jax.numpy as jnp
from jax.experimental import pallas as pl
from jax.experimental.pallas import tpu as pltpu

_TILE = 7168


def _proj_kernel(x_ref, rm_ref, o_ref):
    o_ref[0] = jax.lax.dot_general(
        x_ref[0],
        rm_ref[0],
        dimension_numbers=(((1,), (0,)), ((), ())),
        precision=jax.lax.Precision.DEFAULT,
        preferred_element_type=jnp.float32,
    )


def kernel(inputs, random_matrices):
    n, c, h, w = inputs.shape
    hw = h * w
    m = random_matrices.shape[2]
    xt = inputs.reshape(n, c, hw).transpose(0, 2, 1)

    tile = _TILE if hw % _TILE == 0 else hw
    grid = (n, hw // tile)

    return pl.pallas_call(
        _proj_kernel,
        grid=grid,
        in_specs=[
            pl.BlockSpec((1, tile, c), lambda b, t: (b, t, 0)),
            pl.BlockSpec((1, c, m), lambda b, t: (b, 0, 0)),
        ],
        out_specs=pl.BlockSpec((1, tile, m), lambda b, t: (b, t, 0)),
        out_shape=jax.ShapeDtypeStruct((n, hw, m), jnp.float32),
        compiler_params=pltpu.CompilerParams(
            dimension_semantics=("parallel", "parallel"),
        ),
    )(xt, random_matrices)
